# baseline (speedup 1.0000x reference)
import functools

import jax
import jax.numpy as jnp
from jax import lax
from jax.experimental import pallas as pl
from jax.experimental.pallas import tpu as pltpu
from jax.experimental.pallas import tpu_sc as plsc

N_NODES = 100000
N_EDGES = 1600000
NC, NS = 2, 16
NW = NC * NS
NP = 100352
RB = NP // 128
EP = 1605632
ER = EP // 128
RPW = ER // NW
RPS = ER // NS
NPT = NP // NS
GK = 14
GKE = GK * 128
NG2W = RPW // (2 * GK)
GK3 = 7
GKE3 = GK3 * 128
NG2S = RPS // (2 * GK3)
GK1 = 28
GKE1 = GK1 * 128
NG1 = RPW // GK1

_mesh = plsc.VectorSubcoreMesh(core_axis_name="c", subcore_axis_name="s")
_sc_params = pltpu.CompilerParams(use_tc_tiling_on_sc=False)


@functools.partial(
    pl.kernel, mesh=_mesh, compiler_params=_sc_params,
    out_type=jax.ShapeDtypeStruct((NC, NP), jnp.float32),
    scratch_types=[
        pltpu.VMEM((GKE1,), jnp.int32),
        pltpu.VMEM((GKE1,), jnp.float32),
        pltpu.VMEM_SHARED((NP,), jnp.float32),
    ],
)
def _deg_sc(dst_hbm, ones_hbm, init_hbm, out_hbm, didx, ones_v, acc):
    c = lax.axis_index("c")
    s = lax.axis_index("s")
    wid = s * NC + c
    pltpu.sync_copy(ones_hbm, ones_v)
    pltpu.sync_copy(init_hbm.at[c].at[pl.ds(s * NPT, NPT)],
                    acc.at[pl.ds(s * NPT, NPT)])
    plsc.subcore_barrier()

    def group(gi, carry):
        base = (wid * RPW + gi * GK1) * 128
        pltpu.sync_copy(dst_hbm.at[pl.ds(base, GKE1)], didx)
        pltpu.sync_copy(ones_v, acc.at[didx], add=True)
        return carry

    lax.fori_loop(0, NG1, group, 0)
    plsc.subcore_barrier()
    pltpu.sync_copy(acc.at[pl.ds(s * NPT, NPT)],
                    out_hbm.at[c].at[pl.ds(s * NPT, NPT)])


@functools.partial(
    pl.kernel, mesh=_mesh, compiler_params=_sc_params,
    out_type=jax.ShapeDtypeStruct((NC, NP, 4), jnp.float32),
    scratch_types=[
        pltpu.VMEM((2, GKE), jnp.int32),
        pltpu.VMEM((2, GKE), jnp.int32),
        pltpu.VMEM((GKE, 4), jnp.float32),
        pltpu.VMEM((GKE, 4), jnp.float32),
        pltpu.VMEM_SHARED((NP, 4), jnp.float32),
        pltpu.VMEM_SHARED((NP, 4), jnp.float32),
        pltpu.SemaphoreType.DMA,
    ],
)
def _prop1_sc(src_hbm, dst_hbm, xs_hbm, init_hbm, out_hbm,
              sidx, didx, rows_a, rows_b, tbl, acc, gsem):
    c = lax.axis_index("c")
    s = lax.axis_index("s")
    wid = s * NC + c
    pltpu.sync_copy(xs_hbm.at[pl.ds(s * NPT, NPT)], tbl.at[pl.ds(s * NPT, NPT)])
    pltpu.sync_copy(init_hbm.at[c].at[pl.ds(s * NPT, NPT)],
                    acc.at[pl.ds(s * NPT, NPT)])
    plsc.subcore_barrier()

    def pair(m, carry):
        base = (wid * RPW + m * 2 * GK) * 128
        pltpu.sync_copy(src_hbm.at[pl.ds(base, GKE)], sidx.at[0])
        pltpu.sync_copy(dst_hbm.at[pl.ds(base, GKE)], didx.at[0])
        ga = pltpu.async_copy(tbl.at[sidx.at[0]], rows_a, gsem)
        pltpu.sync_copy(src_hbm.at[pl.ds(base + GKE, GKE)], sidx.at[1])
        pltpu.sync_copy(dst_hbm.at[pl.ds(base + GKE, GKE)], didx.at[1])
        ga.wait()
        gb = pltpu.async_copy(tbl.at[sidx.at[1]], rows_b, gsem)
        pltpu.sync_copy(rows_a, acc.at[didx.at[0]], add=True)
        gb.wait()
        pltpu.sync_copy(rows_b, acc.at[didx.at[1]], add=True)
        return carry

    lax.fori_loop(0, NG2W, pair, 0)
    plsc.subcore_barrier()
    pltpu.sync_copy(acc.at[pl.ds(s * NPT, NPT)],
                    out_hbm.at[c].at[pl.ds(s * NPT, NPT)])


@functools.partial(
    pl.kernel, mesh=_mesh, compiler_params=_sc_params,
    out_type=jax.ShapeDtypeStruct((8, NP, 8), jnp.float32),
    scratch_types=[
        pltpu.VMEM((2, GKE3), jnp.int32),
        pltpu.VMEM((2, GKE3), jnp.int32),
        pltpu.VMEM((GKE3, 8), jnp.float32),
        pltpu.VMEM((GKE3, 8), jnp.float32),
        pltpu.VMEM_SHARED((NP, 8), jnp.float32),
        pltpu.VMEM_SHARED((NP, 8), jnp.float32),
        pltpu.SemaphoreType.DMA,
    ],
)
def _prop2_sc(src_hbm, dst_hbm, gs_hbm, out_hbm,
              sidx, didx, rows_a, rows_b, tbl, acc, gsem):
    c = lax.axis_index("c")
    s = lax.axis_index("s")
    for k in range(4):
        chunk = c * 4 + k
        pltpu.sync_copy(gs_hbm.at[chunk].at[pl.ds(s * NPT, NPT)],
                        tbl.at[pl.ds(s * NPT, NPT)])
        pltpu.sync_copy(gs_hbm.at[chunk].at[pl.ds(s * NPT, NPT)],
                        acc.at[pl.ds(s * NPT, NPT)])
        plsc.subcore_barrier()

        def pair(m, carry):
            base = (s * RPS + m * 2 * GK3) * 128
            pltpu.sync_copy(src_hbm.at[pl.ds(base, GKE3)], sidx.at[0])
            pltpu.sync_copy(dst_hbm.at[pl.ds(base, GKE3)], didx.at[0])
            ga = pltpu.async_copy(tbl.at[sidx.at[0]], rows_a, gsem)
            pltpu.sync_copy(src_hbm.at[pl.ds(base + GKE3, GKE3)], sidx.at[1])
            pltpu.sync_copy(dst_hbm.at[pl.ds(base + GKE3, GKE3)], didx.at[1])
            ga.wait()
            gb = pltpu.async_copy(tbl.at[sidx.at[1]], rows_b, gsem)
            pltpu.sync_copy(rows_a, acc.at[didx.at[0]], add=True)
            gb.wait()
            pltpu.sync_copy(rows_b, acc.at[didx.at[1]], add=True)
            return carry

        lax.fori_loop(0, NG2S, pair, 0)
        plsc.subcore_barrier()
        pltpu.sync_copy(acc.at[pl.ds(s * NPT, NPT)],
                        out_hbm.at[chunk].at[pl.ds(s * NPT, NPT)])
        plsc.subcore_barrier()


def _t1_body(degp_ref, xt_ref, dis_ref, xst_ref):
    deg = degp_ref[0] + degp_ref[1]
    d = lax.rsqrt(deg)
    dis_ref[...] = d
    xst_ref[...] = xt_ref[...] * d[None, :, :]


def _t1(deg_partial, x_t):
    return pl.pallas_call(
        _t1_body,
        out_shape=(
            jax.ShapeDtypeStruct((RB, 128), jnp.float32),
            jax.ShapeDtypeStruct((2, RB, 128), jnp.float32),
        ),
    )(deg_partial, x_t)


_BN = 1024
_GN = NP // _BN


def _t2_body(tp_ref, dis_ref, w1_ref, b1_ref, w2_ref, gs_ref):
    d = dis_ref[...]
    t = (tp_ref[0] + tp_ref[1]) * d
    h1 = t[:, 0:1] * w1_ref[0:1, :] + t[:, 1:2] * w1_ref[1:2, :] + b1_ref[...]
    h1 = jnp.maximum(h1, 0.0)
    g = jnp.dot(h1, w2_ref[...], preferred_element_type=jnp.float32)
    gs_ref[...] = g * d


def _t2(tp, dis_col, W1, b1, W2):
    return pl.pallas_call(
        _t2_body,
        grid=(_GN,),
        in_specs=[
            pl.BlockSpec((NC, _BN, 4), lambda i: (0, i, 0)),
            pl.BlockSpec((_BN, 1), lambda i: (i, 0)),
            pl.BlockSpec((2, 128), lambda i: (0, 0)),
            pl.BlockSpec((1, 128), lambda i: (0, 0)),
            pl.BlockSpec((128, 64), lambda i: (0, 0)),
        ],
        out_specs=pl.BlockSpec((_BN, 64), lambda i: (i, 0)),
        out_shape=jax.ShapeDtypeStruct((NP, 64), jnp.float32),
    )(tp, dis_col, W1, b1, W2)


def _t3_body(p_ref, dis_ref, b2_ref, wp_ref, bp_ref, out_ref):
    h2 = jnp.maximum(p_ref[...] * dis_ref[...] + b2_ref[...], 0.0)
    o = jnp.dot(h2, wp_ref[...], preferred_element_type=jnp.float32)
    o = o + bp_ref[...]
    out_ref[...] = 1.0 / (1.0 + jnp.exp(-o))


def _t3(p_nodes, dis_col, b2, Wp, bp):
    return pl.pallas_call(
        _t3_body,
        grid=(_GN,),
        in_specs=[
            pl.BlockSpec((_BN, 64), lambda i: (i, 0)),
            pl.BlockSpec((_BN, 1), lambda i: (i, 0)),
            pl.BlockSpec((1, 64), lambda i: (0, 0)),
            pl.BlockSpec((64, 1), lambda i: (0, 0)),
            pl.BlockSpec((1, 1), lambda i: (0, 0)),
        ],
        out_specs=pl.BlockSpec((_BN, 1), lambda i: (i, 0)),
        out_shape=jax.ShapeDtypeStruct((NP, 1), jnp.float32),
    )(p_nodes, dis_col, b2, Wp, bp)


def kernel(x, edge_index, W1, b1, W2, b2, Wp, bp):
    src = edge_index[0].astype(jnp.int32)
    dst = edge_index[1].astype(jnp.int32)
    npad = EP - N_EDGES
    pad_idx = N_NODES + (jnp.arange(npad, dtype=jnp.int32) % (NP - N_NODES))
    src_p = jnp.concatenate([src, pad_idx])
    dst_p = jnp.concatenate([dst, pad_idx])

    deg_init = jnp.stack([jnp.ones((NP,), jnp.float32),
                          jnp.zeros((NP,), jnp.float32)])
    ones_e = jnp.ones((GKE1,), jnp.float32)
    deg_partial = _deg_sc(dst_p, ones_e, deg_init)

    x_t = jnp.pad(x.T, ((0, 0), (0, NP - N_NODES))).reshape(2, RB, 128)
    dis, xs_t = _t1(deg_partial.reshape(NC, RB, 128), x_t)

    xs4 = jnp.pad(xs_t.reshape(2, NP).T, ((0, 0), (0, 2)))
    prop1_init = jnp.stack([xs4, jnp.zeros((NP, 4), jnp.float32)])
    tp = _prop1_sc(src_p, dst_p, xs4, prop1_init)

    dis_col = dis.reshape(NP, 1)
    gs = _t2(tp, dis_col, W1, b1.reshape(1, 128), W2)
    gs8 = gs.reshape(NP, 8, 8).transpose(1, 0, 2)

    p = _prop2_sc(src_p, dst_p, gs8)
    p_nodes = p.transpose(1, 0, 2).reshape(NP, 64)

    out = _t3(p_nodes, dis_col, b2.reshape(1, 64), Wp, bp.reshape(1, 1))
    return out[:N_NODES, 0]


# speedup vs baseline: 1.2327x; 1.2327x over previous
import functools

import jax
import jax.numpy as jnp
from jax import lax
from jax.experimental import pallas as pl
from jax.experimental.pallas import tpu as pltpu
from jax.experimental.pallas import tpu_sc as plsc

N_NODES = 100000
N_EDGES = 1600000
NC, NS = 2, 16
NW = NC * NS
NP = 100352
RB = NP // 128
EP = 1622016
ER = EP // 128
RPW = ER // NW
RPS = ER // NS
NPT = NP // NS
GK = 9
GKE = GK * 128
NG2W = RPW // (2 * GK)
GK3 = 6
GKE3 = GK3 * 128
NG2S = RPS // (2 * GK3)
GK1 = 18
GKE1 = GK1 * 128
NG1 = RPW // GK1

_mesh = plsc.VectorSubcoreMesh(core_axis_name="c", subcore_axis_name="s")
_sc_params = pltpu.CompilerParams(use_tc_tiling_on_sc=False)


@functools.partial(
    pl.kernel, mesh=_mesh, compiler_params=_sc_params,
    out_type=jax.ShapeDtypeStruct((NC, NP), jnp.float32),
    scratch_types=[
        pltpu.VMEM((GKE1,), jnp.int32),
        pltpu.VMEM((GKE1,), jnp.float32),
        pltpu.VMEM_SHARED((NP,), jnp.float32),
    ],
)
def _deg_sc(dst_hbm, ones_hbm, init_hbm, out_hbm, didx, ones_v, acc):
    c = lax.axis_index("c")
    s = lax.axis_index("s")
    wid = s * NC + c
    pltpu.sync_copy(ones_hbm, ones_v)
    pltpu.sync_copy(init_hbm.at[c].at[pl.ds(s * NPT, NPT)],
                    acc.at[pl.ds(s * NPT, NPT)])
    plsc.subcore_barrier()

    def group(gi, carry):
        base = (wid * RPW + gi * GK1) * 128
        pltpu.sync_copy(dst_hbm.at[pl.ds(base, GKE1)], didx)
        pltpu.sync_copy(ones_v, acc.at[didx], add=True)
        return carry

    lax.fori_loop(0, NG1, group, 0)
    plsc.subcore_barrier()
    pltpu.sync_copy(acc.at[pl.ds(s * NPT, NPT)],
                    out_hbm.at[c].at[pl.ds(s * NPT, NPT)])


@functools.partial(
    pl.kernel, mesh=_mesh, compiler_params=_sc_params,
    out_type=jax.ShapeDtypeStruct((NC, NP, 4), jnp.float32),
    scratch_types=[
        pltpu.VMEM((2, GKE), jnp.int32),
        pltpu.VMEM((2, GKE), jnp.int32),
        pltpu.VMEM((GKE, 4), jnp.float32),
        pltpu.VMEM((GKE, 4), jnp.float32),
        pltpu.VMEM_SHARED((NP, 4), jnp.float32),
        pltpu.VMEM_SHARED((NP, 4), jnp.float32),
        pltpu.SemaphoreType.DMA,
    ],
)
def _prop1_sc(src_hbm, dst_hbm, xs_hbm, init_hbm, out_hbm,
              sidx, didx, rows_a, rows_b, tbl, acc, gsem):
    c = lax.axis_index("c")
    s = lax.axis_index("s")
    wid = s * NC + c
    pltpu.sync_copy(xs_hbm.at[pl.ds(s * NPT, NPT)], tbl.at[pl.ds(s * NPT, NPT)])
    pltpu.sync_copy(init_hbm.at[c].at[pl.ds(s * NPT, NPT)],
                    acc.at[pl.ds(s * NPT, NPT)])
    plsc.subcore_barrier()

    def pair(m, carry):
        base = (wid * RPW + m * 2 * GK) * 128
        pltpu.sync_copy(src_hbm.at[pl.ds(base, GKE)], sidx.at[0])
        pltpu.sync_copy(dst_hbm.at[pl.ds(base, GKE)], didx.at[0])
        ga = pltpu.async_copy(tbl.at[sidx.at[0]], rows_a, gsem)
        pltpu.sync_copy(src_hbm.at[pl.ds(base + GKE, GKE)], sidx.at[1])
        pltpu.sync_copy(dst_hbm.at[pl.ds(base + GKE, GKE)], didx.at[1])
        ga.wait()
        gb = pltpu.async_copy(tbl.at[sidx.at[1]], rows_b, gsem)
        pltpu.sync_copy(rows_a, acc.at[didx.at[0]], add=True)
        gb.wait()
        pltpu.sync_copy(rows_b, acc.at[didx.at[1]], add=True)
        return carry

    lax.fori_loop(0, NG2W, pair, 0)
    plsc.subcore_barrier()
    pltpu.sync_copy(acc.at[pl.ds(s * NPT, NPT)],
                    out_hbm.at[c].at[pl.ds(s * NPT, NPT)])


@functools.partial(
    pl.kernel, mesh=_mesh, compiler_params=_sc_params,
    out_type=jax.ShapeDtypeStruct((4, NP, 16), jnp.float32),
    scratch_types=[
        pltpu.VMEM((2, 2, GKE3), jnp.int32),
        pltpu.VMEM((GKE3, 16), jnp.float32),
        pltpu.VMEM((GKE3, 16), jnp.float32),
        pltpu.VMEM_SHARED((NP, 16), jnp.float32),
        pltpu.SemaphoreType.DMA,
    ],
)
def _prop2_sc(src_hbm, dst_hbm, gs_hbm, out_hbm,
              idx, rows_a, rows_b, acc, gsem):
    c = lax.axis_index("c")
    s = lax.axis_index("s")
    for k in range(2):
        chunk = c * 2 + k
        pltpu.sync_copy(gs_hbm.at[chunk].at[pl.ds(s * NPT, NPT)],
                        acc.at[pl.ds(s * NPT, NPT)])
        plsc.subcore_barrier()

        def pair(m, carry):
            base = (s * RPS + m * 2 * GK3) * 128
            pltpu.sync_copy(src_hbm.at[pl.ds(base, GKE3)], idx.at[0].at[0])
            pltpu.sync_copy(dst_hbm.at[pl.ds(base, GKE3)], idx.at[0].at[1])
            ga = pltpu.async_copy(gs_hbm.at[chunk].at[idx.at[0].at[0]],
                                  rows_a, gsem)
            pltpu.sync_copy(src_hbm.at[pl.ds(base + GKE3, GKE3)],
                            idx.at[1].at[0])
            pltpu.sync_copy(dst_hbm.at[pl.ds(base + GKE3, GKE3)],
                            idx.at[1].at[1])
            ga.wait()
            gb = pltpu.async_copy(gs_hbm.at[chunk].at[idx.at[1].at[0]],
                                  rows_b, gsem)
            pltpu.sync_copy(rows_a, acc.at[idx.at[0].at[1]], add=True)
            gb.wait()
            pltpu.sync_copy(rows_b, acc.at[idx.at[1].at[1]], add=True)
            return carry

        lax.fori_loop(0, NG2S, pair, 0)
        plsc.subcore_barrier()
        pltpu.sync_copy(acc.at[pl.ds(s * NPT, NPT)],
                        out_hbm.at[chunk].at[pl.ds(s * NPT, NPT)])
        plsc.subcore_barrier()


def _t1_body(degp_ref, xt_ref, dis_ref, xst_ref):
    deg = degp_ref[0] + degp_ref[1]
    d = lax.rsqrt(deg)
    dis_ref[...] = d
    xst_ref[...] = xt_ref[...] * d[None, :, :]


def _t1(deg_partial, x_t):
    return pl.pallas_call(
        _t1_body,
        out_shape=(
            jax.ShapeDtypeStruct((RB, 128), jnp.float32),
            jax.ShapeDtypeStruct((2, RB, 128), jnp.float32),
        ),
    )(deg_partial, x_t)


_BN = 1024
_GN = NP // _BN


def _t2_body(tp_ref, dis_ref, w1_ref, b1_ref, w2_ref, gs_ref):
    d = dis_ref[...]
    t = (tp_ref[0] + tp_ref[1]) * d
    h1 = t[:, 0:1] * w1_ref[0:1, :] + t[:, 1:2] * w1_ref[1:2, :] + b1_ref[...]
    h1 = jnp.maximum(h1, 0.0)
    g = jnp.dot(h1, w2_ref[...], preferred_element_type=jnp.float32)
    gs_ref[...] = g * d


def _t2(tp, dis_col, W1, b1, W2):
    return pl.pallas_call(
        _t2_body,
        grid=(_GN,),
        in_specs=[
            pl.BlockSpec((NC, _BN, 4), lambda i: (0, i, 0)),
            pl.BlockSpec((_BN, 1), lambda i: (i, 0)),
            pl.BlockSpec((2, 128), lambda i: (0, 0)),
            pl.BlockSpec((1, 128), lambda i: (0, 0)),
            pl.BlockSpec((128, 64), lambda i: (0, 0)),
        ],
        out_specs=pl.BlockSpec((_BN, 64), lambda i: (i, 0)),
        out_shape=jax.ShapeDtypeStruct((NP, 64), jnp.float32),
    )(tp, dis_col, W1, b1, W2)


def _t3_body(p_ref, dis_ref, b2_ref, wp_ref, bp_ref, out_ref):
    h2 = jnp.maximum(p_ref[...] * dis_ref[...] + b2_ref[...], 0.0)
    o = jnp.dot(h2, wp_ref[...], preferred_element_type=jnp.float32)
    o = o + bp_ref[...]
    out_ref[...] = 1.0 / (1.0 + jnp.exp(-o))


def _t3(p_nodes, dis_col, b2, Wp, bp):
    return pl.pallas_call(
        _t3_body,
        grid=(_GN,),
        in_specs=[
            pl.BlockSpec((_BN, 64), lambda i: (i, 0)),
            pl.BlockSpec((_BN, 1), lambda i: (i, 0)),
            pl.BlockSpec((1, 64), lambda i: (0, 0)),
            pl.BlockSpec((64, 1), lambda i: (0, 0)),
            pl.BlockSpec((1, 1), lambda i: (0, 0)),
        ],
        out_specs=pl.BlockSpec((_BN, 1), lambda i: (i, 0)),
        out_shape=jax.ShapeDtypeStruct((NP, 1), jnp.float32),
    )(p_nodes, dis_col, b2, Wp, bp)


def kernel(x, edge_index, W1, b1, W2, b2, Wp, bp):
    src = edge_index[0].astype(jnp.int32)
    dst = edge_index[1].astype(jnp.int32)
    npad = EP - N_EDGES
    pad_idx = N_NODES + (jnp.arange(npad, dtype=jnp.int32) % (NP - N_NODES))
    src_p = jnp.concatenate([src, pad_idx])
    dst_p = jnp.concatenate([dst, pad_idx])

    deg_init = jnp.stack([jnp.ones((NP,), jnp.float32),
                          jnp.zeros((NP,), jnp.float32)])
    ones_e = jnp.ones((GKE1,), jnp.float32)
    deg_partial = _deg_sc(dst_p, ones_e, deg_init)

    x_t = jnp.pad(x.T, ((0, 0), (0, NP - N_NODES))).reshape(2, RB, 128)
    dis, xs_t = _t1(deg_partial.reshape(NC, RB, 128), x_t)

    xs4 = jnp.pad(xs_t.reshape(2, NP).T, ((0, 0), (0, 2)))
    prop1_init = jnp.stack([xs4, jnp.zeros((NP, 4), jnp.float32)])
    tp = _prop1_sc(src_p, dst_p, xs4, prop1_init)

    dis_col = dis.reshape(NP, 1)
    gs = _t2(tp, dis_col, W1, b1.reshape(1, 128), W2)
    gs4 = gs.reshape(NP, 4, 16).transpose(1, 0, 2)

    p = _prop2_sc(src_p, dst_p, gs4)
    p_nodes = p.transpose(1, 0, 2).reshape(NP, 64)

    out = _t3(p_nodes, dis_col, b2.reshape(1, 64), Wp, bp.reshape(1, 1))
    return out[:N_NODES, 0]
